# baseline (speedup 1.0000x reference)
import functools

import jax
import jax.numpy as jnp
from jax import lax
from jax.experimental import pallas as pl
from jax.experimental.pallas import tpu as pltpu
from jax.experimental.pallas import tpu_sc as plsc

B = 16384
D = 64
HID = 16
V = 1_000_000

_info = plsc.get_sparse_core_info()
NC, NS = _info.num_cores, _info.num_subcores
NW = NC * NS
L = 16

CW = 512
NFULL = (V // CW // NW) * NW
CPW = NFULL // NW
VMAIN = V // CW * CW

NG = B // L
ROWS_PAD = B + 2048

_mesh = plsc.VectorSubcoreMesh(core_axis_name="c", subcore_axis_name="s")


@functools.partial(
    pl.kernel,
    mesh=_mesh,
    out_type=jax.ShapeDtypeStruct((ROWS_PAD, 2 * D), jnp.float32),
    scratch_types=[
        pltpu.VMEM((NG // 8, 8 * L), jnp.int32),
        pltpu.VMEM((B + L,), jnp.int32),
        pltpu.VMEM((B + L,), jnp.int32),
        pltpu.VMEM((D, CW), jnp.float32),
        pltpu.VMEM((L, 2 * D), jnp.float32),
        pltpu.SemaphoreType.DMA,
    ],
    compiler_params=pltpu.CompilerParams(needs_layout_passes=False),
)
def _sc_gather(idx_hbm, table_t_hbm, rows_hbm, idx_v, list_r, list_j,
               chunk_v, rowbuf, sem):
    wid = lax.axis_index("s") * NC + lax.axis_index("c")
    lo = wid * (CPW * CW)
    n_chunks = jnp.where(wid == NW - 1, CPW + 1, CPW)
    hi = lo + n_chunks * CW

    pltpu.sync_copy(idx_hbm, idx_v)

    def bin_row(row, cnt):
        for k in range(8):
            rv = idx_v[row, pl.ds(k * L, L)]
            jv = (row * 8 + k) * L + lax.iota(jnp.int32, L)
            m = (rv >= lo) & (rv < hi)
            plsc.store_compressed(list_r.at[pl.ds(cnt, L)], rv, mask=m)
            plsc.store_compressed(list_j.at[pl.ds(cnt, L)], jv, mask=m)
            cnt = cnt + plsc.all_reduce_population_count(m)[0]
        return cnt

    cnt = lax.fori_loop(0, NG // 8, bin_row, 0)
    ngroups = (cnt + L - 1) // L

    def do_chunk(c, _):
        col0 = pl.multiple_of(lo + c * CW, CW)
        pltpu.sync_copy(table_t_hbm.at[:, pl.ds(col0, CW)], chunk_v)

        def do_group(gi, _g):
            rv = list_r[pl.ds(gi * L, L)]
            jv = list_j[pl.ds(gi * L, L)]
            valid = gi * L + lax.iota(jnp.int32, L) < cnt
            m2 = (rv >= col0) & (rv < col0 + CW) & valid

            @pl.when(plsc.all_reduce_population_count(m2)[0] > 0)
            def _extract():
                rel = jnp.clip(rv - col0, 0, CW - 1)
                jsafe = jnp.where(m2, jv, B + lax.iota(jnp.int32, L))
                lanes = lax.iota(jnp.int32, L)
                for cc in range(D):
                    val = plsc.load_gather(
                        chunk_v, [jnp.full((L,), cc, dtype=jnp.int32), rel]
                    )
                    plsc.store_scatter(rowbuf, [lanes, jnp.full((L,), cc, dtype=jnp.int32)], val)
                pltpu.async_copy(rowbuf, rows_hbm.at[jsafe], sem).wait()

            return _g

        lax.fori_loop(0, ngroups, do_group, 0)
        return _

    lax.fori_loop(0, n_chunks, do_chunk, 0)


def _mlp_body(rows_ref, idx_ref, tail_ref, w1_ref, b1_ref, w2_ref, b2_ref,
              val_ref, embt_ref):
    emb = rows_ref[:, :D]
    idx = idx_ref[...]
    rel = idx - VMAIN
    onehot = (rel == lax.broadcasted_iota(jnp.int32, (1, D), 1)).astype(
        jnp.float32
    )
    tail_rows = jnp.dot(
        onehot,
        tail_ref[...],
        preferred_element_type=jnp.float32,
        precision=lax.Precision.HIGHEST,
    )
    emb = jnp.where(idx >= VMAIN, tail_rows, emb)
    embt_ref[...] = emb.T
    h = jnp.dot(emb, w1_ref[...], preferred_element_type=jnp.float32)
    h = jnp.maximum(h + b1_ref[...], 0.0)
    val_ref[...] = (
        jnp.dot(h, w2_ref[...], preferred_element_type=jnp.float32) + b2_ref[...]
    )


_BBLK = 2048


def _tc_mlp(rows, idx, tail, w1, b1, w2, b2):
    grid = (B // _BBLK,)
    return pl.pallas_call(
        _mlp_body,
        grid=grid,
        in_specs=[
            pl.BlockSpec((_BBLK, 2 * D), lambda i: (i, 0)),
            pl.BlockSpec((_BBLK, 1), lambda i: (i, 0)),
            pl.BlockSpec((D, D), lambda i: (0, 0)),
            pl.BlockSpec((D, HID), lambda i: (0, 0)),
            pl.BlockSpec((1, HID), lambda i: (0, 0)),
            pl.BlockSpec((HID, 1), lambda i: (0, 0)),
            pl.BlockSpec((1, 1), lambda i: (0, 0)),
        ],
        out_specs=[
            pl.BlockSpec((_BBLK, 1), lambda i: (i, 0)),
            pl.BlockSpec((D, _BBLK), lambda i: (0, i)),
        ],
        out_shape=[
            jax.ShapeDtypeStruct((B, 1), jnp.float32),
            jax.ShapeDtypeStruct((D, B), jnp.float32),
        ],
    )(rows, idx, tail, w1, b1, w2, b2)


def kernel(states, emb_table, W1, b1, W2, b2):
    idx2d = states.reshape(NG // 8, 8 * L)
    rows = _sc_gather(idx2d, emb_table.T)
    tail = lax.slice(emb_table, (VMAIN, 0), (V, D))
    values, emb_t = _tc_mlp(
        rows, states, tail, W1, b1.reshape(1, HID), W2, b2.reshape(1, 1)
    )
    return emb_t.T, values


# speedup vs baseline: 1.0156x; 1.0156x over previous
import functools

import jax
import jax.numpy as jnp
from jax import lax
from jax.experimental import pallas as pl
from jax.experimental.pallas import tpu as pltpu
from jax.experimental.pallas import tpu_sc as plsc

B = 16384
D = 64
HID = 16
V = 1_000_000

_info = plsc.get_sparse_core_info()
NC, NS = _info.num_cores, _info.num_subcores
NW = NC * NS
L = 16

CW = 512
NFULL = (V // CW // NW) * NW
CPW = NFULL // NW
VMAIN = V // CW * CW

NG = B // L
ROWS_PAD = B + 2048

_mesh = plsc.VectorSubcoreMesh(core_axis_name="c", subcore_axis_name="s")


@functools.partial(
    pl.kernel,
    mesh=_mesh,
    out_type=jax.ShapeDtypeStruct((ROWS_PAD, 2 * D), jnp.float32),
    scratch_types=[
        pltpu.VMEM((NG // 8, 8 * L), jnp.int32),
        pltpu.VMEM((B + L,), jnp.int32),
        pltpu.VMEM((B + L,), jnp.int32),
        pltpu.VMEM((2, D, CW), jnp.float32),
        pltpu.VMEM((L, 2 * D), jnp.float32),
        pltpu.SemaphoreType.DMA,
        pltpu.SemaphoreType.DMA,
        pltpu.SemaphoreType.DMA,
    ],
    compiler_params=pltpu.CompilerParams(needs_layout_passes=False),
)
def _sc_gather(idx_hbm, table_t_hbm, rows_hbm, idx_v, list_r, list_j,
               chunk_v, rowbuf, sem, sem0, sem1):
    wid = lax.axis_index("s") * NC + lax.axis_index("c")
    lo = wid * (CPW * CW)
    n_chunks = jnp.where(wid == NW - 1, CPW + 1, CPW)
    hi = lo + n_chunks * CW

    pltpu.sync_copy(idx_hbm, idx_v)

    def bin_row(row, cnt):
        for k in range(8):
            rv = idx_v[row, pl.ds(k * L, L)]
            jv = (row * 8 + k) * L + lax.iota(jnp.int32, L)
            m = (rv >= lo) & (rv < hi)
            plsc.store_compressed(list_r.at[pl.ds(cnt, L)], rv, mask=m)
            plsc.store_compressed(list_j.at[pl.ds(cnt, L)], jv, mask=m)
            cnt = cnt + plsc.all_reduce_population_count(m)[0]
        return cnt

    cnt = lax.fori_loop(0, NG // 8, bin_row, 0)
    ngroups = (cnt + L - 1) // L

    def fire(c, par, s):
        col0 = pl.multiple_of(lo + c * CW, CW)
        for i in range(D // 8):
            pltpu.async_copy(
                table_t_hbm.at[pl.ds(i * 8, 8), pl.ds(col0, CW)],
                chunk_v.at[par, pl.ds(i * 8, 8)],
                s,
            )

    def drain(par, s):
        for i in range(D // 8):
            pltpu.make_async_copy(
                table_t_hbm.at[pl.ds(0, 8), pl.ds(0, CW)],
                chunk_v.at[par, pl.ds(i * 8, 8)],
                s,
            ).wait()

    fire(0, 0, sem0)

    def do_chunk(c, _):
        par = c & 1
        more = c + 1 < n_chunks

        @pl.when(more & (par == 0))
        def _f1():
            fire(c + 1, 1, sem1)

        @pl.when(more & (par == 1))
        def _f0():
            fire(c + 1, 0, sem0)

        @pl.when(par == 0)
        def _d0():
            drain(0, sem0)

        @pl.when(par == 1)
        def _d1():
            drain(1, sem1)

        col0 = lo + c * CW
        parv = jnp.full((L,), par, dtype=jnp.int32)

        def do_group(gi, _g):
            rv = list_r[pl.ds(gi * L, L)]
            jv = list_j[pl.ds(gi * L, L)]
            valid = gi * L + lax.iota(jnp.int32, L) < cnt
            m2 = (rv >= col0) & (rv < col0 + CW) & valid

            @pl.when(plsc.all_reduce_population_count(m2)[0] > 0)
            def _extract():
                rel = jnp.clip(rv - col0, 0, CW - 1)
                jsafe = jnp.where(m2, jv, B + lax.iota(jnp.int32, L))
                lanes = lax.iota(jnp.int32, L)
                for cc in range(D):
                    val = plsc.load_gather(
                        chunk_v,
                        [parv, jnp.full((L,), cc, dtype=jnp.int32), rel],
                    )
                    plsc.store_scatter(rowbuf, [lanes, jnp.full((L,), cc, dtype=jnp.int32)], val)
                pltpu.async_copy(rowbuf, rows_hbm.at[jsafe], sem).wait()

            return _g

        lax.fori_loop(0, ngroups, do_group, 0)
        return _

    lax.fori_loop(0, n_chunks, do_chunk, 0)


def _mlp_body(rows_ref, idx_ref, tail_ref, w1_ref, b1_ref, w2_ref, b2_ref,
              val_ref, embt_ref):
    emb = rows_ref[:, :D]
    idx = idx_ref[...]
    rel = idx - VMAIN
    onehot = (rel == lax.broadcasted_iota(jnp.int32, (1, D), 1)).astype(
        jnp.float32
    )
    tail_rows = jnp.dot(
        onehot,
        tail_ref[...],
        preferred_element_type=jnp.float32,
        precision=lax.Precision.HIGHEST,
    )
    emb = jnp.where(idx >= VMAIN, tail_rows, emb)
    embt_ref[...] = emb.T
    h = jnp.dot(emb, w1_ref[...], preferred_element_type=jnp.float32)
    h = jnp.maximum(h + b1_ref[...], 0.0)
    val_ref[...] = (
        jnp.dot(h, w2_ref[...], preferred_element_type=jnp.float32) + b2_ref[...]
    )


_BBLK = 2048


def _tc_mlp(rows, idx, tail, w1, b1, w2, b2):
    grid = (B // _BBLK,)
    return pl.pallas_call(
        _mlp_body,
        grid=grid,
        in_specs=[
            pl.BlockSpec((_BBLK, 2 * D), lambda i: (i, 0)),
            pl.BlockSpec((_BBLK, 1), lambda i: (i, 0)),
            pl.BlockSpec((D, D), lambda i: (0, 0)),
            pl.BlockSpec((D, HID), lambda i: (0, 0)),
            pl.BlockSpec((1, HID), lambda i: (0, 0)),
            pl.BlockSpec((HID, 1), lambda i: (0, 0)),
            pl.BlockSpec((1, 1), lambda i: (0, 0)),
        ],
        out_specs=[
            pl.BlockSpec((_BBLK, 1), lambda i: (i, 0)),
            pl.BlockSpec((D, _BBLK), lambda i: (0, i)),
        ],
        out_shape=[
            jax.ShapeDtypeStruct((B, 1), jnp.float32),
            jax.ShapeDtypeStruct((D, B), jnp.float32),
        ],
    )(rows, idx, tail, w1, b1, w2, b2)


def kernel(states, emb_table, W1, b1, W2, b2):
    idx2d = states.reshape(NG // 8, 8 * L)
    rows = _sc_gather(idx2d, emb_table.T)
    tail = lax.slice(emb_table, (VMAIN, 0), (V, D))
    values, emb_t = _tc_mlp(
        rows, states, tail, W1, b1.reshape(1, HID), W2, b2.reshape(1, 1)
    )
    return emb_t.T, values


# speedup vs baseline: 1.0333x; 1.0174x over previous
import functools

import jax
import jax.numpy as jnp
from jax import lax
from jax.experimental import pallas as pl
from jax.experimental.pallas import tpu as pltpu
from jax.experimental.pallas import tpu_sc as plsc

B = 16384
D = 64
HID = 16
V = 1_000_000

_info = plsc.get_sparse_core_info()
NC, NS = _info.num_cores, _info.num_subcores
NW = NC * NS
L = 16

CW = 512
NFULL = (V // CW // NW) * NW
CPW = NFULL // NW
VMAIN = V // CW * CW

NG = B // L
ROWS_PAD = B + 2048

_mesh = plsc.VectorSubcoreMesh(core_axis_name="c", subcore_axis_name="s")


@functools.partial(
    pl.kernel,
    mesh=_mesh,
    out_type=jax.ShapeDtypeStruct((ROWS_PAD, 2 * D), jnp.float32),
    scratch_types=[
        pltpu.VMEM((NG // 8, 8 * L), jnp.int32),
        pltpu.VMEM((B + L,), jnp.int32),
        pltpu.VMEM((B + L,), jnp.int32),
        pltpu.VMEM((2, D, CW), jnp.float32),
        pltpu.VMEM((4, L, 2 * D), jnp.float32),
        pltpu.SemaphoreType.DMA,
        pltpu.SemaphoreType.DMA,
        pltpu.SemaphoreType.DMA,
    ],
    compiler_params=pltpu.CompilerParams(needs_layout_passes=False),
)
def _sc_gather(idx_hbm, table_t_hbm, rows_hbm, idx_v, list_r, list_j,
               chunk_v, rowbuf, sem, sem0, sem1):
    wid = lax.axis_index("s") * NC + lax.axis_index("c")
    lo = wid * (CPW * CW)
    n_chunks = jnp.where(wid == NW - 1, CPW + 1, CPW)
    hi = lo + n_chunks * CW

    pltpu.sync_copy(idx_hbm, idx_v)

    def bin_row(row, cnt):
        for k in range(8):
            rv = idx_v[row, pl.ds(k * L, L)]
            jv = (row * 8 + k) * L + lax.iota(jnp.int32, L)
            m = (rv >= lo) & (rv < hi)
            plsc.store_compressed(list_r.at[pl.ds(cnt, L)], rv, mask=m)
            plsc.store_compressed(list_j.at[pl.ds(cnt, L)], jv, mask=m)
            cnt = cnt + plsc.all_reduce_population_count(m)[0]
        return cnt

    cnt = lax.fori_loop(0, NG // 8, bin_row, 0)
    ngroups = (cnt + L - 1) // L

    def fire(c, par, s):
        col0 = pl.multiple_of(lo + c * CW, CW)
        for i in range(D // 8):
            pltpu.async_copy(
                table_t_hbm.at[pl.ds(i * 8, 8), pl.ds(col0, CW)],
                chunk_v.at[par, pl.ds(i * 8, 8)],
                s,
            )

    def drain(par, s):
        for i in range(D // 8):
            pltpu.make_async_copy(
                table_t_hbm.at[pl.ds(0, 8), pl.ds(0, CW)],
                chunk_v.at[par, pl.ds(i * 8, 8)],
                s,
            ).wait()

    fire(0, 0, sem0)

    def do_chunk(c, _):
        par = c & 1
        more = c + 1 < n_chunks

        @pl.when(more & (par == 0))
        def _f1():
            fire(c + 1, 1, sem1)

        @pl.when(more & (par == 1))
        def _f0():
            fire(c + 1, 0, sem0)

        @pl.when(par == 0)
        def _d0():
            drain(0, sem0)

        @pl.when(par == 1)
        def _d1():
            drain(1, sem1)

        col0 = lo + c * CW
        parv = jnp.full((L,), par, dtype=jnp.int32)

        def do_group(gi, scnt):
            rv = list_r[pl.ds(gi * L, L)]
            jv = list_j[pl.ds(gi * L, L)]
            valid = gi * L + lax.iota(jnp.int32, L) < cnt
            m2 = (rv >= col0) & (rv < col0 + CW) & valid
            active = plsc.all_reduce_population_count(m2)[0] > 0

            @pl.when(active)
            def _extract():
                @pl.when(scnt >= 3)
                def _lazy():
                    pltpu.make_async_copy(
                        rows_hbm.at[pl.ds(B, L)], rowbuf.at[0], sem
                    ).wait()

                slot = scnt & 3
                rel = jnp.clip(rv - col0, 0, CW - 1)
                jsafe = jnp.where(m2, jv, B + lax.iota(jnp.int32, L))
                lanes = lax.iota(jnp.int32, L)
                slotv = jnp.full((L,), slot, dtype=jnp.int32)
                for cc in range(D):
                    val = plsc.load_gather(
                        chunk_v,
                        [parv, jnp.full((L,), cc, dtype=jnp.int32), rel],
                    )
                    plsc.store_scatter(
                        rowbuf,
                        [slotv, lanes, jnp.full((L,), cc, dtype=jnp.int32)],
                        val,
                    )
                pltpu.async_copy(rowbuf.at[slot], rows_hbm.at[jsafe], sem)

            return scnt + active.astype(jnp.int32)

        return lax.fori_loop(0, ngroups, do_group, _)

    scnt = lax.fori_loop(0, n_chunks, do_chunk, 0)
    for k in range(1, 4):
        @pl.when(scnt >= k)
        def _final_drain():
            pltpu.make_async_copy(
                rows_hbm.at[pl.ds(B, L)], rowbuf.at[0], sem
            ).wait()


def _mlp_body(rows_ref, idx_ref, tail_ref, w1_ref, b1_ref, w2_ref, b2_ref,
              val_ref, embt_ref):
    emb = rows_ref[:, :D]
    idx = idx_ref[...]
    rel = idx - VMAIN
    onehot = (rel == lax.broadcasted_iota(jnp.int32, (1, D), 1)).astype(
        jnp.float32
    )
    tail_rows = jnp.dot(
        onehot,
        tail_ref[...],
        preferred_element_type=jnp.float32,
        precision=lax.Precision.HIGHEST,
    )
    emb = jnp.where(idx >= VMAIN, tail_rows, emb)
    embt_ref[...] = emb.T
    h = jnp.dot(emb, w1_ref[...], preferred_element_type=jnp.float32)
    h = jnp.maximum(h + b1_ref[...], 0.0)
    val_ref[...] = (
        jnp.dot(h, w2_ref[...], preferred_element_type=jnp.float32) + b2_ref[...]
    )


_BBLK = 2048


def _tc_mlp(rows, idx, tail, w1, b1, w2, b2):
    grid = (B // _BBLK,)
    return pl.pallas_call(
        _mlp_body,
        grid=grid,
        in_specs=[
            pl.BlockSpec((_BBLK, 2 * D), lambda i: (i, 0)),
            pl.BlockSpec((_BBLK, 1), lambda i: (i, 0)),
            pl.BlockSpec((D, D), lambda i: (0, 0)),
            pl.BlockSpec((D, HID), lambda i: (0, 0)),
            pl.BlockSpec((1, HID), lambda i: (0, 0)),
            pl.BlockSpec((HID, 1), lambda i: (0, 0)),
            pl.BlockSpec((1, 1), lambda i: (0, 0)),
        ],
        out_specs=[
            pl.BlockSpec((_BBLK, 1), lambda i: (i, 0)),
            pl.BlockSpec((D, _BBLK), lambda i: (0, i)),
        ],
        out_shape=[
            jax.ShapeDtypeStruct((B, 1), jnp.float32),
            jax.ShapeDtypeStruct((D, B), jnp.float32),
        ],
    )(rows, idx, tail, w1, b1, w2, b2)


def kernel(states, emb_table, W1, b1, W2, b2):
    idx2d = states.reshape(NG // 8, 8 * L)
    rows = _sc_gather(idx2d, emb_table.T)
    tail = lax.slice(emb_table, (VMAIN, 0), (V, D))
    values, emb_t = _tc_mlp(
        rows, states, tail, W1, b1.reshape(1, HID), W2, b2.reshape(1, 1)
    )
    return emb_t.T, values


# speedup vs baseline: 2.4132x; 2.3355x over previous
import functools

import jax
import jax.numpy as jnp
from jax import lax
from jax.experimental import pallas as pl
from jax.experimental.pallas import tpu as pltpu
from jax.experimental.pallas import tpu_sc as plsc

B = 16384
D = 64
HID = 16

_info = plsc.get_sparse_core_info()
NC, NS = _info.num_cores, _info.num_subcores
NW = NC * NS
B_PER_W = B // NW

_mesh = plsc.VectorSubcoreMesh(core_axis_name="c", subcore_axis_name="s")

G = 16
NG = B_PER_W // G


@functools.partial(
    pl.kernel,
    mesh=_mesh,
    out_type=jax.ShapeDtypeStruct((B, D), jnp.float32),
    scratch_types=[
        pltpu.VMEM((B_PER_W,), jnp.int32),
        pltpu.VMEM((B_PER_W, D), jnp.float32),
        pltpu.SemaphoreType.DMA,
    ],
)
def _sc_gather(idx_hbm, table_hbm, emb_hbm, idx_v, rows_v, sem):
    wid = lax.axis_index("s") * NC + lax.axis_index("c")
    jbase = wid * B_PER_W
    pltpu.sync_copy(idx_hbm.at[wid], idx_v)

    def body(g, _):
        vec = idx_v[pl.ds(g * G, G)]
        base = g * G
        for l in range(G):
            s = vec[l]
            pltpu.async_copy(
                table_hbm.at[pl.ds(s, 1)], rows_v.at[pl.ds(base + l, 1)], sem
            )

        @pl.when(g >= 1)
        def _wait():
            for l in range(G):
                pltpu.make_async_copy(
                    table_hbm.at[pl.ds(0, 1)],
                    rows_v.at[pl.ds(base - G + l, 1)],
                    sem,
                ).wait()

        return 0

    lax.fori_loop(0, NG, body, 0)
    for l in range(G):
        pltpu.make_async_copy(
            table_hbm.at[pl.ds(0, 1)],
            rows_v.at[pl.ds((NG - 1) * G + l, 1)],
            sem,
        ).wait()
    pltpu.sync_copy(rows_v, emb_hbm.at[pl.ds(jbase, B_PER_W)])


def _mlp_body(emb_ref, w1_ref, b1_ref, w2_ref, b2_ref, out_ref):
    h = jnp.dot(emb_ref[...], w1_ref[...], preferred_element_type=jnp.float32)
    h = jnp.maximum(h + b1_ref[...], 0.0)
    out_ref[...] = (
        jnp.dot(h, w2_ref[...], preferred_element_type=jnp.float32) + b2_ref[...]
    )


_BBLK = 2048


def _tc_mlp(emb, w1, b1, w2, b2):
    grid = (B // _BBLK,)
    return pl.pallas_call(
        _mlp_body,
        grid=grid,
        in_specs=[
            pl.BlockSpec((_BBLK, D), lambda i: (i, 0)),
            pl.BlockSpec((D, HID), lambda i: (0, 0)),
            pl.BlockSpec((1, HID), lambda i: (0, 0)),
            pl.BlockSpec((HID, 1), lambda i: (0, 0)),
            pl.BlockSpec((1, 1), lambda i: (0, 0)),
        ],
        out_specs=pl.BlockSpec((_BBLK, 1), lambda i: (i, 0)),
        out_shape=jax.ShapeDtypeStruct((B, 1), jnp.float32),
    )(emb, w1, b1, w2, b2)


def kernel(states, emb_table, W1, b1, W2, b2):
    idx = states.reshape(NW, B_PER_W)
    emb = _sc_gather(idx, emb_table)
    values = _tc_mlp(emb, W1, b1.reshape(1, HID), W2, b2.reshape(1, 1))
    return emb, values
